# Initial kernel scaffold; baseline (speedup 1.0000x reference)
import jax
import jax.numpy as jnp
from jax.experimental import pallas as pl


def kernel(preds, targets):
    raise NotImplementedError("write your pallas kernel here")


# speedup vs baseline: 4.5904x; 4.5904x over previous
import functools

import jax
import jax.numpy as jnp
from jax import lax
from jax.experimental import pallas as pl
from jax.experimental.pallas import tpu as pltpu
from jax.experimental.pallas import tpu_sc as plsc

N = 32768
C = 64
K = N // 2
L = 16
NV = N // L
KV = K // L
TWIN = 2048
NWIN = N // TWIN
MPAD = K + L

_VARK = K * (K + 1) / 12.0
_SCALE = float(1.0 / (K * (_VARK + 1e-8)))
_MHALF = (K - 1) / 2.0

def _isum(v):
    return jnp.sum(v.astype(jnp.int32))


def _ic_grid_kernel(p_hbm, t_hbm, out_hbm, pbuf, twin, mk, mv, bk, bv, hist,
                    obuf):
    cid = lax.axis_index("c")
    sid = lax.axis_index("s")
    wid = sid * 2 + cid
    ii = lax.iota(jnp.int32, L)

    def zero_hist(nbins):
        def z(ch, c):
            hist[pl.ds(ch * L, L)] = jnp.zeros((L,), jnp.int32)
            return c
        lax.fori_loop(0, nbins // L, z, 0)

    def scan_hist(nbins, r):
        def sc(ch, carry):
            tot, bfound, cb = carry
            h = hist[pl.ds(ch * L, L)]
            cs = plsc.cumsum(h)
            cum = cs + tot
            found = cum >= r
            lane = _isum(jnp.where(found, jnp.int32(0), jnp.int32(1)))
            anyf = lane < L
            excl = cum - h
            cbc = _isum(jnp.where(ii == lane, excl, jnp.int32(0)))
            hit = (bfound < 0) & anyf
            bfound = jnp.where(hit, ch * L + lane, bfound)
            cb = jnp.where(hit, cbc, cb)
            return tot + _isum(h), bfound, cb
        _, b, cb = lax.fori_loop(0, nbins // L, sc,
                                 (jnp.int32(0), jnp.int32(-1), jnp.int32(0)))
        return b, cb

    def hist_pass(nvec, digit_fn, src):
        def hp(v, c):
            d = digit_fn(src[pl.ds(v * L, L)])
            cnt, lastm = plsc.scan_count(d)
            plsc.addupdate_scatter(hist, [d], cnt, mask=lastm)
            return c
        lax.fori_loop(0, nvec, hp, 0)

    def masked_hist_pass(nvec, digit_fn, mask_fn, src):
        def hp(v, c):
            k = src[pl.ds(v * L, L)]
            m = mask_fn(k)
            d = digit_fn(k)
            cnt, lastm = plsc.scan_count(d, mask=m)
            plsc.addupdate_scatter(hist, [d], cnt, mask=lastm)
            return c
        lax.fori_loop(0, nvec, hp, 0)

    def exclusive_prefix(nbins):
        def pf(ch, tot):
            h = hist[pl.ds(ch * L, L)]
            cs = plsc.cumsum(h)
            hist[pl.ds(ch * L, L)] = cs - h + tot
            return tot + _isum(h)
        lax.fori_loop(0, nbins // L, pf, jnp.int32(0))

    def radix_pass(nbins, digit_fn, src_k, dst_k, dst_v, src_v=None,
                   gen_val=None):
        zero_hist(nbins)
        hist_pass(KV, digit_fn, src_k)
        exclusive_prefix(nbins)

        def sp(v, c):
            k = src_k[pl.ds(v * L, L)]
            if gen_val is not None:
                val = gen_val(v)
            else:
                val = src_v[pl.ds(v * L, L)]
            d = digit_fn(k)
            cnt, lastm = plsc.scan_count(d)
            base = plsc.load_gather(hist, [d])
            pos = base + cnt - 1
            plsc.store_scatter(dst_k, [pos], k)
            plsc.store_scatter(dst_v, [pos], val)
            plsc.addupdate_scatter(hist, [d], cnt, mask=lastm)
            return c
        lax.fori_loop(0, KV, sp, 0)

    d_lo11 = lambda k: k & 0x7FF
    d_mid11 = lambda k: lax.shift_right_logical(k, 11) & 0x7FF
    d_top10 = lambda k: lax.shift_right_logical(k, 22) ^ 0x200

    def do_column(col):
        pltpu.sync_copy(p_hbm.at[col], pbuf)

        zero_hist(2048)

        def pa(v, c):
            x = pbuf[pl.ds(v * L, L)]
            s = lax.shift_right_arithmetic(x, 31)
            key = ~(x ^ lax.shift_right_logical(s, 1))
            pbuf[pl.ds(v * L, L)] = key
            d = lax.shift_right_logical(key, 21) ^ 0x400
            cnt, lastm = plsc.scan_count(d)
            plsc.addupdate_scatter(hist, [d], cnt, mask=lastm)
            return c
        lax.fori_loop(0, NV, pa, 0)
        b1, cb1 = scan_hist(2048, jnp.int32(K))
        r2 = K - cb1
        tb1 = (b1 ^ 0x400) << 21

        zero_hist(2048)
        masked_hist_pass(
            NV,
            lambda k: lax.shift_right_logical(k, 10) & 0x7FF,
            lambda k: lax.shift_right_logical(k, 21) == lax.shift_right_logical(tb1, 21),
            pbuf)
        b2, cb2 = scan_hist(2048, r2)
        r3 = r2 - cb2
        tb2 = tb1 | (b2 << 10)

        zero_hist(1024)
        masked_hist_pass(
            NV,
            lambda k: k & 0x3FF,
            lambda k: lax.shift_right_logical(k, 10) == lax.shift_right_logical(tb2, 10),
            pbuf)
        b3, cb3 = scan_hist(1024, r3)
        e_need = r3 - cb3
        tb = tb2 | b3

        def cw(w, carry):
            wofs, eqs = carry
            pltpu.sync_copy(t_hbm.at[col, pl.ds(w * TWIN, TWIN)], twin)

            def cv(v, carry2):
                wofs, eqs = carry2
                key = pbuf[pl.ds((w * (TWIN // L) + v) * L, L)]
                tx = twin[pl.ds(v * L, L)]
                ts = lax.shift_right_arithmetic(tx, 31)
                kt = tx ^ lax.shift_right_logical(ts, 1)
                lt = key < tb
                eq = key == tb
                ec = plsc.cumsum(jnp.where(eq, jnp.int32(1), jnp.int32(0)))
                take = eq & ((ec + eqs) <= e_need)
                member = lt | take
                plsc.store_compressed(mk.at[pl.ds(wofs, L)], key, mask=member)
                plsc.store_compressed(mv.at[pl.ds(wofs, L)], kt, mask=member)
                return wofs + _isum(member), eqs + _isum(eq)
            return lax.fori_loop(0, TWIN // L, cv, (wofs, eqs))
        lax.fori_loop(0, NWIN, cw, (jnp.int32(0), jnp.int32(0)))

        radix_pass(2048, d_lo11, mk, bk, bv, src_v=mv)
        radix_pass(2048, d_mid11, bk, mk, mv, src_v=bv)
        radix_pass(1024, d_top10, mk, bk, bv, src_v=mv)

        radix_pass(2048, d_lo11, bv, mk, mv, gen_val=lambda v: v * L + ii)
        radix_pass(2048, d_mid11, mk, bk, bv, src_v=mv)
        radix_pass(1024, d_top10, bk, mk, mv, src_v=bv)

        def cv2(v, acc):
            sig = mv[pl.ds(v * L, L)].astype(jnp.float32)
            j = (v * L + ii).astype(jnp.float32)
            return acc + (j - _MHALF) * (_MHALF - sig)
        acc = lax.fori_loop(0, KV, cv2, jnp.zeros((L,), jnp.float32))
        ic = jnp.sum(acc) * _SCALE
        obuf[...] = jnp.full((L,), ic, jnp.float32)
        pltpu.sync_copy(obuf, out_hbm.at[col])

    for rep in range(2):
        do_column(wid + 32 * rep)


@functools.partial(
    pl.kernel,
    out_type=jax.ShapeDtypeStruct((C, L), jnp.float32),
    mesh=plsc.VectorSubcoreMesh(core_axis_name="c", subcore_axis_name="s"),
    compiler_params=pltpu.CompilerParams(needs_layout_passes=False),
    scratch_types=[
        pltpu.VMEM((N,), jnp.int32),
        pltpu.VMEM((TWIN,), jnp.int32),
        pltpu.VMEM((MPAD,), jnp.int32),
        pltpu.VMEM((MPAD,), jnp.int32),
        pltpu.VMEM((MPAD,), jnp.int32),
        pltpu.VMEM((MPAD,), jnp.int32),
        pltpu.VMEM((2048,), jnp.int32),
        pltpu.VMEM((L,), jnp.float32),
    ],
)
def _ic_sc(p_hbm, t_hbm, out_hbm, pbuf, twin, mk, mv, bk, bv, hist, obuf):
    _ic_grid_kernel(p_hbm, t_hbm, out_hbm, pbuf, twin, mk, mv, bk, bv, hist,
                    obuf)


def kernel(preds, targets):
    pT = lax.bitcast_convert_type(preds.T, jnp.int32)
    tT = lax.bitcast_convert_type(targets.T, jnp.int32)
    out = _ic_sc(pT, tT)
    return out[:, 0]


# speedup vs baseline: 4.8073x; 1.0473x over previous
import functools

import jax
import jax.numpy as jnp
from jax import lax
from jax.experimental import pallas as pl
from jax.experimental.pallas import tpu as pltpu
from jax.experimental.pallas import tpu_sc as plsc

N = 32768
C = 64
K = N // 2
L = 16
NV = N // L
KV = K // L
TWIN = 2048
NWIN = N // TWIN
MPAD = K + L

_VARK = K * (K + 1) / 12.0
_SCALE = float(1.0 / (K * (_VARK + 1e-8)))
_MHALF = (K - 1) / 2.0

def _isum(v):
    return jnp.sum(v.astype(jnp.int32))


def _ic_grid_kernel(p_hbm, t_hbm, out_hbm, pbuf, twin, mk, mv, bk, bv, hist,
                    obuf):
    cid = lax.axis_index("c")
    sid = lax.axis_index("s")
    wid = sid * 2 + cid
    ii = lax.iota(jnp.int32, L)

    def zero_hist(nbins):
        def z(ch, c):
            hist[pl.ds(ch * L, L)] = jnp.zeros((L,), jnp.int32)
            return c
        lax.fori_loop(0, nbins // L, z, 0, unroll=8)

    def scan_hist(nbins, r):
        def sc(ch, carry):
            tot, bfound, cb = carry
            h = hist[pl.ds(ch * L, L)]
            cs = plsc.cumsum(h)
            cum = cs + tot
            found = cum >= r
            lane = _isum(jnp.where(found, jnp.int32(0), jnp.int32(1)))
            anyf = lane < L
            excl = cum - h
            cbc = _isum(jnp.where(ii == lane, excl, jnp.int32(0)))
            hit = (bfound < 0) & anyf
            bfound = jnp.where(hit, ch * L + lane, bfound)
            cb = jnp.where(hit, cbc, cb)
            return tot + _isum(h), bfound, cb
        _, b, cb = lax.fori_loop(0, nbins // L, sc,
                                 (jnp.int32(0), jnp.int32(-1), jnp.int32(0)))
        return b, cb

    def hist_pass(nvec, digit_fn, src):
        def hp(v, c):
            d = digit_fn(src[pl.ds(v * L, L)])
            cnt, lastm = plsc.scan_count(d)
            plsc.addupdate_scatter(hist, [d], cnt, mask=lastm)
            return c
        lax.fori_loop(0, nvec, hp, 0, unroll=4)

    def masked_hist_pass(nvec, digit_fn, mask_fn, src):
        def hp(v, c):
            k = src[pl.ds(v * L, L)]
            m = mask_fn(k)
            d = digit_fn(k)
            cnt, lastm = plsc.scan_count(d, mask=m)
            plsc.addupdate_scatter(hist, [d], cnt, mask=lastm)
            return c
        lax.fori_loop(0, nvec, hp, 0, unroll=4)

    def exclusive_prefix(nbins):
        def pf(ch, tot):
            h = hist[pl.ds(ch * L, L)]
            cs = plsc.cumsum(h)
            hist[pl.ds(ch * L, L)] = cs - h + tot
            return tot + _isum(h)
        lax.fori_loop(0, nbins // L, pf, jnp.int32(0), unroll=2)

    def radix_pass(nbins, digit_fn, src_k, dst_k, dst_v, src_v=None,
                   gen_val=None):
        zero_hist(nbins)
        hist_pass(KV, digit_fn, src_k)
        exclusive_prefix(nbins)

        def sp(v, c):
            k = src_k[pl.ds(v * L, L)]
            if gen_val is not None:
                val = gen_val(v)
            else:
                val = src_v[pl.ds(v * L, L)]
            d = digit_fn(k)
            cnt, lastm = plsc.scan_count(d)
            base = plsc.load_gather(hist, [d])
            pos = base + cnt - 1
            plsc.store_scatter(dst_k, [pos], k)
            plsc.store_scatter(dst_v, [pos], val)
            plsc.addupdate_scatter(hist, [d], cnt, mask=lastm)
            return c
        lax.fori_loop(0, KV, sp, 0, unroll=4)

    d_lo11 = lambda k: k & 0x7FF
    d_mid11 = lambda k: lax.shift_right_logical(k, 11) & 0x7FF
    d_top10 = lambda k: lax.shift_right_logical(k, 22) ^ 0x200

    def do_column(col):
        pltpu.sync_copy(p_hbm.at[col], pbuf)

        zero_hist(2048)

        def pa(v, c):
            x = pbuf[pl.ds(v * L, L)]
            s = lax.shift_right_arithmetic(x, 31)
            key = ~(x ^ lax.shift_right_logical(s, 1))
            pbuf[pl.ds(v * L, L)] = key
            d = lax.shift_right_logical(key, 21) ^ 0x400
            cnt, lastm = plsc.scan_count(d)
            plsc.addupdate_scatter(hist, [d], cnt, mask=lastm)
            return c
        lax.fori_loop(0, NV, pa, 0, unroll=4)
        b1, cb1 = scan_hist(2048, jnp.int32(K))
        r2 = K - cb1
        tb1 = (b1 ^ 0x400) << 21

        zero_hist(2048)
        masked_hist_pass(
            NV,
            lambda k: lax.shift_right_logical(k, 10) & 0x7FF,
            lambda k: lax.shift_right_logical(k, 21) == lax.shift_right_logical(tb1, 21),
            pbuf)
        b2, cb2 = scan_hist(2048, r2)
        r3 = r2 - cb2
        tb2 = tb1 | (b2 << 10)

        zero_hist(1024)
        masked_hist_pass(
            NV,
            lambda k: k & 0x3FF,
            lambda k: lax.shift_right_logical(k, 10) == lax.shift_right_logical(tb2, 10),
            pbuf)
        b3, cb3 = scan_hist(1024, r3)
        e_need = r3 - cb3
        tb = tb2 | b3

        def cw(w, carry):
            wofs, eqs = carry
            pltpu.sync_copy(t_hbm.at[col, pl.ds(w * TWIN, TWIN)], twin)

            def cv(v, carry2):
                wofs, eqs = carry2
                key = pbuf[pl.ds((w * (TWIN // L) + v) * L, L)]
                tx = twin[pl.ds(v * L, L)]
                ts = lax.shift_right_arithmetic(tx, 31)
                kt = tx ^ lax.shift_right_logical(ts, 1)
                lt = key < tb
                eq = key == tb
                ec = plsc.cumsum(jnp.where(eq, jnp.int32(1), jnp.int32(0)))
                take = eq & ((ec + eqs) <= e_need)
                member = lt | take
                plsc.store_compressed(mk.at[pl.ds(wofs, L)], key, mask=member)
                plsc.store_compressed(mv.at[pl.ds(wofs, L)], kt, mask=member)
                return wofs + _isum(member), eqs + _isum(eq)
            return lax.fori_loop(0, TWIN // L, cv, (wofs, eqs), unroll=4)
        lax.fori_loop(0, NWIN, cw, (jnp.int32(0), jnp.int32(0)))

        radix_pass(2048, d_lo11, mk, bk, bv, src_v=mv)
        radix_pass(2048, d_mid11, bk, mk, mv, src_v=bv)
        radix_pass(1024, d_top10, mk, bk, bv, src_v=mv)

        radix_pass(2048, d_lo11, bv, mk, mv, gen_val=lambda v: v * L + ii)
        radix_pass(2048, d_mid11, mk, bk, bv, src_v=mv)
        radix_pass(1024, d_top10, bk, mk, mv, src_v=bv)

        def cv2(v, acc):
            sig = mv[pl.ds(v * L, L)].astype(jnp.float32)
            j = (v * L + ii).astype(jnp.float32)
            return acc + (j - _MHALF) * (_MHALF - sig)
        acc = lax.fori_loop(0, KV, cv2, jnp.zeros((L,), jnp.float32), unroll=4)
        ic = jnp.sum(acc) * _SCALE
        obuf[...] = jnp.full((L,), ic, jnp.float32)
        pltpu.sync_copy(obuf, out_hbm.at[col])

    for rep in range(2):
        do_column(wid + 32 * rep)


@functools.partial(
    pl.kernel,
    out_type=jax.ShapeDtypeStruct((C, L), jnp.float32),
    mesh=plsc.VectorSubcoreMesh(core_axis_name="c", subcore_axis_name="s"),
    compiler_params=pltpu.CompilerParams(needs_layout_passes=False),
    scratch_types=[
        pltpu.VMEM((N,), jnp.int32),
        pltpu.VMEM((TWIN,), jnp.int32),
        pltpu.VMEM((MPAD,), jnp.int32),
        pltpu.VMEM((MPAD,), jnp.int32),
        pltpu.VMEM((MPAD,), jnp.int32),
        pltpu.VMEM((MPAD,), jnp.int32),
        pltpu.VMEM((2048,), jnp.int32),
        pltpu.VMEM((L,), jnp.float32),
    ],
)
def _ic_sc(p_hbm, t_hbm, out_hbm, pbuf, twin, mk, mv, bk, bv, hist, obuf):
    _ic_grid_kernel(p_hbm, t_hbm, out_hbm, pbuf, twin, mk, mv, bk, bv, hist,
                    obuf)


def kernel(preds, targets):
    pT = lax.bitcast_convert_type(preds.T, jnp.int32)
    tT = lax.bitcast_convert_type(targets.T, jnp.int32)
    out = _ic_sc(pT, tT)
    return out[:, 0]


# speedup vs baseline: 4.8407x; 1.0069x over previous
import functools

import jax
import jax.numpy as jnp
from jax import lax
from jax.experimental import pallas as pl
from jax.experimental.pallas import tpu as pltpu
from jax.experimental.pallas import tpu_sc as plsc

N = 32768
C = 64
K = N // 2
L = 16
NV = N // L
KV = K // L
SS = 4
NCH = NV // SS
KCH = KV // SS
TWIN = 2048
NWIN = N // TWIN
MPAD = K + L

_VARK = K * (K + 1) / 12.0
_SCALE = float(1.0 / (K * (_VARK + 1e-8)))
_MHALF = (K - 1) / 2.0


def _isum(v):
    return jnp.sum(v.astype(jnp.int32))


def _f32(v):
    return v.astype(jnp.float32)


def _ic_grid_kernel(p_hbm, t_hbm, out_hbm, pbuf, twin, mk, mv, bk, bv, obuf,
                    *hists):
    cid = lax.axis_index("c")
    sid = lax.axis_index("s")
    wid = sid * 2 + cid
    ii = lax.iota(jnp.int32, L)

    def zero_hist(nbins):
        def z(ch, c):
            for s in range(SS):
                hists[s][pl.ds(ch * L, L)] = jnp.zeros((L,), jnp.int32)
            return c
        lax.fori_loop(0, nbins // L, z, 0, unroll=4)

    def scan_hist(nbins, r):
        def sc(ch, carry):
            tot, bfound, cb = carry
            h = (hists[0][pl.ds(ch * L, L)] + hists[1][pl.ds(ch * L, L)] +
                 hists[2][pl.ds(ch * L, L)] + hists[3][pl.ds(ch * L, L)])
            cs = plsc.cumsum(h)
            cum = cs + tot
            found = cum >= r
            lane = _isum(jnp.where(found, jnp.int32(0), jnp.int32(1)))
            anyf = lane < L
            excl = cum - h
            cbc = _isum(jnp.where(ii == lane, excl, jnp.int32(0)))
            hit = (bfound < 0) & anyf
            bfound = jnp.where(hit, ch * L + lane, bfound)
            cb = jnp.where(hit, cbc, cb)
            return tot + _isum(h), bfound, cb
        _, b, cb = lax.fori_loop(0, nbins // L, sc,
                                 (jnp.int32(0), jnp.int32(-1), jnp.int32(0)))
        return b, cb

    def hist_pass(nch, digit_fn, src, mask_fn=None):
        def hp(v, c):
            for s in range(SS):
                k = src[pl.ds((s * nch + v) * L, L)]
                d = digit_fn(k)
                if mask_fn is None:
                    cnt, lastm = plsc.scan_count(d)
                else:
                    cnt, lastm = plsc.scan_count(d, mask=mask_fn(k))
                plsc.addupdate_scatter(hists[s], [d], cnt, mask=lastm)
            return c
        lax.fori_loop(0, nch, hp, 0, unroll=2)

    def stream_prefix(nbins):
        def pf(ch, tot):
            hv = [hists[s][pl.ds(ch * L, L)] for s in range(SS)]
            h = hv[0] + hv[1] + hv[2] + hv[3]
            cs = plsc.cumsum(h)
            run = cs - h + tot
            for s in range(SS):
                hists[s][pl.ds(ch * L, L)] = run
                run = run + hv[s]
            return tot + _isum(h)
        lax.fori_loop(0, nbins // L, pf, jnp.int32(0), unroll=2)

    def radix_pass(nbins, digit_fn, src_k, dst_k, dst_v, src_v=None,
                   gen_val=None, fuse_cov=False):
        zero_hist(nbins)
        hist_pass(KCH, digit_fn, src_k)
        stream_prefix(nbins)

        def sp(v, acc):
            for s in range(SS):
                idx = s * KCH + v
                k = src_k[pl.ds(idx * L, L)]
                if gen_val is not None:
                    val = gen_val(idx)
                else:
                    val = src_v[pl.ds(idx * L, L)]
                d = digit_fn(k)
                cnt, lastm = plsc.scan_count(d)
                base = plsc.load_gather(hists[s], [d])
                pos = base + cnt - 1
                plsc.addupdate_scatter(hists[s], [d], cnt, mask=lastm)
                if fuse_cov:
                    acc = acc + (_MHALF - _f32(val)) * (_f32(pos) - _MHALF)
                else:
                    if dst_k is not None:
                        plsc.store_scatter(dst_k, [pos], k)
                    plsc.store_scatter(dst_v, [pos], val)
            return acc
        return lax.fori_loop(0, KCH, sp, jnp.zeros((L,), jnp.float32),
                             unroll=2)

    d_lo11 = lambda k: k & 0x7FF
    d_mid11 = lambda k: lax.shift_right_logical(k, 11) & 0x7FF
    d_top10 = lambda k: lax.shift_right_logical(k, 22) ^ 0x200

    def do_column(col):
        pltpu.sync_copy(p_hbm.at[col], pbuf)

        zero_hist(2048)

        def pa(v, c):
            for s in range(SS):
                x = pbuf[pl.ds((s * NCH + v) * L, L)]
                sg = lax.shift_right_arithmetic(x, 31)
                key = ~(x ^ lax.shift_right_logical(sg, 1))
                pbuf[pl.ds((s * NCH + v) * L, L)] = key
                d = lax.shift_right_logical(key, 21) ^ 0x400
                cnt, lastm = plsc.scan_count(d)
                plsc.addupdate_scatter(hists[s], [d], cnt, mask=lastm)
            return c
        lax.fori_loop(0, NCH, pa, 0, unroll=2)
        b1, cb1 = scan_hist(2048, jnp.int32(K))
        r2 = K - cb1
        tb1 = (b1 ^ 0x400) << 21

        zero_hist(2048)
        hist_pass(
            NCH,
            lambda k: lax.shift_right_logical(k, 10) & 0x7FF,
            pbuf,
            mask_fn=lambda k: lax.shift_right_logical(k, 21)
            == lax.shift_right_logical(tb1, 21))
        b2, cb2 = scan_hist(2048, r2)
        r3 = r2 - cb2
        tb2 = tb1 | (b2 << 10)

        zero_hist(1024)
        hist_pass(
            NCH,
            lambda k: k & 0x3FF,
            pbuf,
            mask_fn=lambda k: lax.shift_right_logical(k, 10)
            == lax.shift_right_logical(tb2, 10))
        b3, cb3 = scan_hist(1024, r3)
        e_need = r3 - cb3
        tb = tb2 | b3

        def cw(w, carry):
            pltpu.sync_copy(t_hbm.at[col, pl.ds(w * TWIN, TWIN)], twin)

            def cv(v, carry2):
                wofs, eqs = carry2
                key = pbuf[pl.ds((w * (TWIN // L) + v) * L, L)]
                tx = twin[pl.ds(v * L, L)]
                ts = lax.shift_right_arithmetic(tx, 31)
                kt = tx ^ lax.shift_right_logical(ts, 1)
                lt = key < tb
                eq = key == tb
                ec = plsc.cumsum(jnp.where(eq, jnp.int32(1), jnp.int32(0)))
                take = eq & ((ec + eqs) <= e_need)
                member = lt | take
                plsc.store_compressed(mk.at[pl.ds(wofs, L)], key, mask=member)
                plsc.store_compressed(mv.at[pl.ds(wofs, L)], kt, mask=member)
                nlt = plsc.all_reduce_population_count(lt)[0]
                neq = plsc.all_reduce_population_count(eq)[0]
                ntake = jnp.clip(e_need - eqs, 0, neq)
                return wofs + nlt + ntake, eqs + neq
            return lax.fori_loop(0, TWIN // L, cv, carry, unroll=2)
        lax.fori_loop(0, NWIN, cw, (jnp.int32(0), jnp.int32(0)))

        radix_pass(2048, d_lo11, mk, bk, bv, src_v=mv)
        radix_pass(2048, d_mid11, bk, mk, mv, src_v=bv)
        radix_pass(1024, d_top10, mk, None, bv, src_v=mv)

        radix_pass(2048, d_lo11, bv, mk, mv, gen_val=lambda v: v * L + ii)
        radix_pass(2048, d_mid11, mk, bk, bv, src_v=mv)
        acc = radix_pass(1024, d_top10, bk, None, None, src_v=bv,
                         fuse_cov=True)

        ic = jnp.sum(acc) * _SCALE
        obuf[...] = jnp.full((L,), ic, jnp.float32)
        pltpu.sync_copy(obuf, out_hbm.at[col])
        return 0

    lax.fori_loop(0, 2, lambda rep, c: do_column(wid + 32 * rep), 0)


@functools.partial(
    pl.kernel,
    out_type=jax.ShapeDtypeStruct((C, L), jnp.float32),
    mesh=plsc.VectorSubcoreMesh(core_axis_name="c", subcore_axis_name="s"),
    compiler_params=pltpu.CompilerParams(needs_layout_passes=False),
    scratch_types=[
        pltpu.VMEM((N,), jnp.int32),
        pltpu.VMEM((TWIN,), jnp.int32),
        pltpu.VMEM((MPAD,), jnp.int32),
        pltpu.VMEM((MPAD,), jnp.int32),
        pltpu.VMEM((MPAD,), jnp.int32),
        pltpu.VMEM((MPAD,), jnp.int32),
        pltpu.VMEM((L,), jnp.float32),
        pltpu.VMEM((2048,), jnp.int32),
        pltpu.VMEM((2048,), jnp.int32),
        pltpu.VMEM((2048,), jnp.int32),
        pltpu.VMEM((2048,), jnp.int32),
    ],
)
def _ic_sc(p_hbm, t_hbm, out_hbm, pbuf, twin, mk, mv, bk, bv, obuf,
           h0, h1, h2, h3):
    _ic_grid_kernel(p_hbm, t_hbm, out_hbm, pbuf, twin, mk, mv, bk, bv, obuf,
                    h0, h1, h2, h3)


def kernel(preds, targets):
    pT = lax.bitcast_convert_type(preds.T, jnp.int32)
    tT = lax.bitcast_convert_type(targets.T, jnp.int32)
    out = _ic_sc(pT, tT)
    return out[:, 0]


# speedup vs baseline: 11.1062x; 2.2943x over previous
import functools

import jax
import jax.numpy as jnp
from jax import lax
from jax.experimental import pallas as pl
from jax.experimental.pallas import tpu as pltpu
from jax.experimental.pallas import tpu_sc as plsc

N = 32768
C = 64
K = N // 2
L = 16
NV = N // L
KV = K // L
TWIN = 2048
NWIN = N // TWIN
MPAD = K + L

_VARK = K * (K + 1) / 12.0
_SCALE = float(1.0 / (K * (_VARK + 1e-8)))
_MHALF = (K - 1) / 2.0


def _isum(v):
    return jnp.sum(v.astype(jnp.int32))


def _f32(v):
    return v.astype(jnp.float32)


def _ic_grid_kernel(p_hbm, t_hbm, out_hbm, pbuf, twin, mk, mv, bk, bv, hist,
                    obuf):
    cid = lax.axis_index("c")
    sid = lax.axis_index("s")
    wid = sid * 2 + cid
    ii = lax.iota(jnp.int32, L)

    def zero_hist(nbins):
        @plsc.parallel_loop(0, nbins // L, unroll=4)
        def _(ch):
            hist[pl.ds(ch * L, L)] = jnp.zeros((L,), jnp.int32)

    def scan_hist(nbins, r):
        def sc(ch, carry):
            tot, bfound, cb = carry
            h = hist[pl.ds(ch * L, L)]
            cs = plsc.cumsum(h)
            cum = cs + tot
            found = cum >= r
            lane = _isum(jnp.where(found, jnp.int32(0), jnp.int32(1)))
            anyf = lane < L
            excl = cum - h
            cbc = _isum(jnp.where(ii == lane, excl, jnp.int32(0)))
            hit = (bfound < 0) & anyf
            bfound = jnp.where(hit, ch * L + lane, bfound)
            cb = jnp.where(hit, cbc, cb)
            return tot + _isum(h), bfound, cb
        _, b, cb = plsc.parallel_loop(
            0, nbins // L, unroll=2,
            carry=(jnp.int32(0), jnp.int32(-1), jnp.int32(0)))(sc)
        return b, cb

    def hist_pass(digit_fn, src, mask_fn=None):
        @plsc.parallel_loop(0, KV, unroll=4)
        def _(v):
            k = src[pl.ds(v * L, L)]
            d = digit_fn(k)
            if mask_fn is None:
                cnt, lastm = plsc.scan_count(d)
            else:
                cnt, lastm = plsc.scan_count(d, mask=mask_fn(k))
            plsc.addupdate_scatter(hist, [d], cnt, mask=lastm)

    def sel_pass(digit_fn, mask_fn):
        @plsc.parallel_loop(0, NV, unroll=4)
        def _(v):
            k = pbuf[pl.ds(v * L, L)]
            d = digit_fn(k)
            cnt, lastm = plsc.scan_count(d, mask=mask_fn(k))
            plsc.addupdate_scatter(hist, [d], cnt, mask=lastm)

    def exclusive_prefix(nbins):
        def pf(ch, tot):
            h = hist[pl.ds(ch * L, L)]
            cs = plsc.cumsum(h)
            hist[pl.ds(ch * L, L)] = cs - h + (tot - 1)
            return tot + _isum(h)
        plsc.parallel_loop(0, nbins // L, unroll=2, carry=jnp.int32(0))(pf)

    def radix_pass(nbins, digit_fn, src_k, dst_k, dst_v, src_v=None,
                   gen_val=None, fuse_cov=False):
        zero_hist(nbins)
        hist_pass(digit_fn, src_k)
        exclusive_prefix(nbins)

        def sp(v, acc):
            k = src_k[pl.ds(v * L, L)]
            if gen_val is not None:
                val = gen_val(v)
            else:
                val = src_v[pl.ds(v * L, L)]
            d = digit_fn(k)
            cnt, lastm = plsc.scan_count(d)
            base = plsc.load_gather(hist, [d])
            pos = base + cnt
            plsc.addupdate_scatter(hist, [d], cnt, mask=lastm)
            if fuse_cov:
                acc = acc + (_MHALF - _f32(val)) * (_f32(pos) - _MHALF)
            else:
                if dst_k is not None:
                    plsc.store_scatter(dst_k, [pos], k)
                plsc.store_scatter(dst_v, [pos], val)
            return acc
        return lax.fori_loop(0, KV, sp, jnp.zeros((L,), jnp.float32),
                             unroll=4)

    d_lo11 = lambda k: k & 0x7FF
    d_mid11 = lambda k: lax.shift_right_logical(k, 11) & 0x7FF
    d_top10 = lambda k: lax.shift_right_logical(k, 22) ^ 0x200

    def do_column(col):
        pltpu.sync_copy(p_hbm.at[col], pbuf)

        zero_hist(2048)

        @plsc.parallel_loop(0, NV, unroll=4)
        def _(v):
            x = pbuf[pl.ds(v * L, L)]
            sg = lax.shift_right_arithmetic(x, 31)
            key = ~(x ^ lax.shift_right_logical(sg, 1))
            pbuf[pl.ds(v * L, L)] = key
            d = lax.shift_right_logical(key, 21) ^ 0x400
            cnt, lastm = plsc.scan_count(d)
            plsc.addupdate_scatter(hist, [d], cnt, mask=lastm)

        b1, cb1 = scan_hist(2048, jnp.int32(K))
        r2 = K - cb1
        tb1 = (b1 ^ 0x400) << 21

        zero_hist(2048)
        sel_pass(
            lambda k: lax.shift_right_logical(k, 10) & 0x7FF,
            lambda k: lax.shift_right_logical(k, 21)
            == lax.shift_right_logical(tb1, 21))
        b2, cb2 = scan_hist(2048, r2)
        r3 = r2 - cb2
        tb2 = tb1 | (b2 << 10)

        zero_hist(1024)
        sel_pass(
            lambda k: k & 0x3FF,
            lambda k: lax.shift_right_logical(k, 10)
            == lax.shift_right_logical(tb2, 10))
        b3, cb3 = scan_hist(1024, r3)
        e_need = r3 - cb3
        tb = tb2 | b3

        def cw(w, carry):
            pltpu.sync_copy(t_hbm.at[col, pl.ds(w * TWIN, TWIN)], twin)

            def cv(v, carry2):
                wofs, eqs = carry2
                key = pbuf[pl.ds((w * (TWIN // L) + v) * L, L)]
                tx = twin[pl.ds(v * L, L)]
                ts = lax.shift_right_arithmetic(tx, 31)
                kt = tx ^ lax.shift_right_logical(ts, 1)
                lt = key < tb
                eq = key == tb
                ec = plsc.cumsum(jnp.where(eq, jnp.int32(1), jnp.int32(0)))
                take = eq & ((ec + eqs) <= e_need)
                member = lt | take
                plsc.store_compressed(mk.at[pl.ds(wofs, L)], key, mask=member)
                plsc.store_compressed(mv.at[pl.ds(wofs, L)], kt, mask=member)
                nlt = plsc.all_reduce_population_count(lt)[0]
                neq = plsc.all_reduce_population_count(eq)[0]
                ntake = jnp.clip(e_need - eqs, 0, neq)
                return wofs + nlt + ntake, eqs + neq
            return plsc.parallel_loop(0, TWIN // L, unroll=4, carry=carry)(cv)
        lax.fori_loop(0, NWIN, cw, (jnp.int32(0), jnp.int32(0)))

        radix_pass(2048, d_lo11, mk, bk, bv, src_v=mv)
        radix_pass(2048, d_mid11, bk, mk, mv, src_v=bv)
        radix_pass(1024, d_top10, mk, None, bv, src_v=mv)

        radix_pass(2048, d_lo11, bv, mk, mv, gen_val=lambda v: v * L + ii)
        radix_pass(2048, d_mid11, mk, bk, bv, src_v=mv)
        acc = radix_pass(1024, d_top10, bk, None, None, src_v=bv,
                         fuse_cov=True)

        ic = jnp.sum(acc) * _SCALE
        obuf[...] = jnp.full((L,), ic, jnp.float32)
        pltpu.sync_copy(obuf, out_hbm.at[col])
        return 0

    lax.fori_loop(0, 2, lambda rep, c: do_column(wid + 32 * rep), 0)


@functools.partial(
    pl.kernel,
    out_type=jax.ShapeDtypeStruct((C, L), jnp.float32),
    mesh=plsc.VectorSubcoreMesh(core_axis_name="c", subcore_axis_name="s"),
    compiler_params=pltpu.CompilerParams(needs_layout_passes=False),
    scratch_types=[
        pltpu.VMEM((N,), jnp.int32),
        pltpu.VMEM((TWIN,), jnp.int32),
        pltpu.VMEM((MPAD,), jnp.int32),
        pltpu.VMEM((MPAD,), jnp.int32),
        pltpu.VMEM((MPAD,), jnp.int32),
        pltpu.VMEM((MPAD,), jnp.int32),
        pltpu.VMEM((2048,), jnp.int32),
        pltpu.VMEM((L,), jnp.float32),
    ],
)
def _ic_sc(p_hbm, t_hbm, out_hbm, pbuf, twin, mk, mv, bk, bv, hist, obuf):
    _ic_grid_kernel(p_hbm, t_hbm, out_hbm, pbuf, twin, mk, mv, bk, bv, hist,
                    obuf)


def kernel(preds, targets):
    pT = lax.bitcast_convert_type(preds.T, jnp.int32)
    tT = lax.bitcast_convert_type(targets.T, jnp.int32)
    out = _ic_sc(pT, tT)
    return out[:, 0]


# speedup vs baseline: 19.0020x; 1.7109x over previous
import functools

import jax
import jax.numpy as jnp
from jax import lax
from jax.experimental import pallas as pl
from jax.experimental.pallas import tpu as pltpu
from jax.experimental.pallas import tpu_sc as plsc

N = 32768
C = 64
K = N // 2
L = 16
NV = N // L
KV = K // L
TWIN = 2048
NWIN = N // TWIN
MPAD = K + L

_VARK = K * (K + 1) / 12.0
_SCALE = float(1.0 / (K * (_VARK + 1e-8)))
_MHALF = (K - 1) / 2.0


def _isum(v):
    return jnp.sum(v.astype(jnp.int32))


def _f32(v):
    return v.astype(jnp.float32)


NQ = 4
KQ = KV // NQ


def _ic_grid_kernel(p_hbm, t_hbm, out_hbm, pbuf, twin, mk, mv, bk, bv, obuf,
                    *hists):
    cid = lax.axis_index("c")
    sid = lax.axis_index("s")
    wid = sid * 2 + cid
    ii = lax.iota(jnp.int32, L)
    hist = hists[0]

    def zero_hist(nbins):
        @plsc.parallel_loop(0, nbins // L, unroll=4)
        def _(ch):
            for q in range(NQ):
                hists[q][pl.ds(ch * L, L)] = jnp.zeros((L,), jnp.int32)

    def scan_hist(nbins, r):
        def sc(ch, carry):
            tot, bfound, cb = carry
            h = hist[pl.ds(ch * L, L)]
            cs = plsc.cumsum(h)
            cum = cs + tot
            found = cum >= r
            lane = _isum(jnp.where(found, jnp.int32(0), jnp.int32(1)))
            anyf = lane < L
            excl = cum - h
            cbc = _isum(jnp.where(ii == lane, excl, jnp.int32(0)))
            hit = (bfound < 0) & anyf
            bfound = jnp.where(hit, ch * L + lane, bfound)
            cb = jnp.where(hit, cbc, cb)
            return tot + _isum(h), bfound, cb
        _, b, cb = plsc.parallel_loop(
            0, nbins // L, unroll=2,
            carry=(jnp.int32(0), jnp.int32(-1), jnp.int32(0)))(sc)
        return b, cb

    def hist_pass(digit_fn, src):
        @plsc.parallel_loop(0, KQ, unroll=2)
        def _(v):
            for q in range(NQ):
                k = src[pl.ds((q * KQ + v) * L, L)]
                d = digit_fn(k)
                cnt, lastm = plsc.scan_count(d)
                plsc.addupdate_scatter(hists[q], [d], cnt, mask=lastm)

    def sel_pass(digit_fn, mask_fn):
        @plsc.parallel_loop(0, NV, unroll=4)
        def _(v):
            k = pbuf[pl.ds(v * L, L)]
            d = digit_fn(k)
            cnt, lastm = plsc.scan_count(d, mask=mask_fn(k))
            plsc.addupdate_scatter(hist, [d], cnt, mask=lastm)

    def exclusive_prefix(nbins):
        def pf(ch, tot):
            hv = [hists[q][pl.ds(ch * L, L)] for q in range(NQ)]
            h = hv[0] + hv[1] + hv[2] + hv[3]
            cs = plsc.cumsum(h)
            run = cs - h + (tot - 1)
            for q in range(NQ):
                hists[q][pl.ds(ch * L, L)] = run
                run = run + hv[q]
            return tot + _isum(h)
        plsc.parallel_loop(0, nbins // L, unroll=2, carry=jnp.int32(0))(pf)

    def radix_pass(nbins, digit_fn, src_k, dst_k, dst_v, src_v=None,
                   gen_val=None, fuse_cov=False):
        zero_hist(nbins)
        hist_pass(digit_fn, src_k)
        exclusive_prefix(nbins)

        def sp(v, acc):
            idx = [q * KQ + v for q in range(NQ)]
            k = [src_k[pl.ds(idx[q] * L, L)] for q in range(NQ)]
            if gen_val is not None:
                val = [gen_val(idx[q]) for q in range(NQ)]
            else:
                val = [src_v[pl.ds(idx[q] * L, L)] for q in range(NQ)]
            d = [digit_fn(k[q]) for q in range(NQ)]
            cl = [plsc.scan_count(d[q]) for q in range(NQ)]
            base = [plsc.load_gather(hists[q], [d[q]]) for q in range(NQ)]
            pos = [base[q] + cl[q][0] for q in range(NQ)]
            for q in range(NQ):
                plsc.addupdate_scatter(hists[q], [d[q]], cl[q][0],
                                       mask=cl[q][1])
            if fuse_cov:
                for q in range(NQ):
                    acc = acc + ((_MHALF - _f32(val[q]))
                                 * (_f32(pos[q]) - _MHALF))
            else:
                for q in range(NQ):
                    if dst_k is not None:
                        plsc.store_scatter(dst_k, [pos[q]], k[q])
                    plsc.store_scatter(dst_v, [pos[q]], val[q])
            return acc
        return lax.fori_loop(0, KQ, sp, jnp.zeros((L,), jnp.float32),
                             unroll=2)

    d_lo11 = lambda k: k & 0x7FF
    d_mid11 = lambda k: lax.shift_right_logical(k, 11) & 0x7FF
    d_top10 = lambda k: lax.shift_right_logical(k, 22) ^ 0x200

    def do_column(col):
        pltpu.sync_copy(p_hbm.at[col], pbuf)

        zero_hist(2048)

        @plsc.parallel_loop(0, NV, unroll=4)
        def _(v):
            x = pbuf[pl.ds(v * L, L)]
            sg = lax.shift_right_arithmetic(x, 31)
            key = ~(x ^ lax.shift_right_logical(sg, 1))
            pbuf[pl.ds(v * L, L)] = key
            d = lax.shift_right_logical(key, 21) ^ 0x400
            cnt, lastm = plsc.scan_count(d)
            plsc.addupdate_scatter(hist, [d], cnt, mask=lastm)

        b1, cb1 = scan_hist(2048, jnp.int32(K))
        r2 = K - cb1
        tb1 = (b1 ^ 0x400) << 21

        zero_hist(2048)
        sel_pass(
            lambda k: lax.shift_right_logical(k, 10) & 0x7FF,
            lambda k: lax.shift_right_logical(k, 21)
            == lax.shift_right_logical(tb1, 21))
        b2, cb2 = scan_hist(2048, r2)
        r3 = r2 - cb2
        tb2 = tb1 | (b2 << 10)

        zero_hist(1024)
        sel_pass(
            lambda k: k & 0x3FF,
            lambda k: lax.shift_right_logical(k, 10)
            == lax.shift_right_logical(tb2, 10))
        b3, cb3 = scan_hist(1024, r3)
        e_need = r3 - cb3
        tb = tb2 | b3

        def cw(w, carry):
            pltpu.sync_copy(t_hbm.at[col, pl.ds(w * TWIN, TWIN)], twin)

            def cv(v, carry2):
                wofs, eqs = carry2
                key = pbuf[pl.ds((w * (TWIN // L) + v) * L, L)]
                tx = twin[pl.ds(v * L, L)]
                ts = lax.shift_right_arithmetic(tx, 31)
                kt = tx ^ lax.shift_right_logical(ts, 1)
                lt = key < tb
                eq = key == tb
                ec = plsc.cumsum(jnp.where(eq, jnp.int32(1), jnp.int32(0)))
                take = eq & ((ec + eqs) <= e_need)
                member = lt | take
                plsc.store_compressed(mk.at[pl.ds(wofs, L)], key, mask=member)
                plsc.store_compressed(mv.at[pl.ds(wofs, L)], kt, mask=member)
                nlt = plsc.all_reduce_population_count(lt)[0]
                neq = plsc.all_reduce_population_count(eq)[0]
                ntake = jnp.clip(e_need - eqs, 0, neq)
                return wofs + nlt + ntake, eqs + neq
            return plsc.parallel_loop(0, TWIN // L, unroll=4, carry=carry)(cv)
        lax.fori_loop(0, NWIN, cw, (jnp.int32(0), jnp.int32(0)))

        radix_pass(2048, d_lo11, mk, bk, bv, src_v=mv)
        radix_pass(2048, d_mid11, bk, mk, mv, src_v=bv)
        radix_pass(1024, d_top10, mk, None, bv, src_v=mv)

        radix_pass(2048, d_lo11, bv, mk, mv, gen_val=lambda v: v * L + ii)
        radix_pass(2048, d_mid11, mk, bk, bv, src_v=mv)
        acc = radix_pass(1024, d_top10, bk, None, None, src_v=bv,
                         fuse_cov=True)

        ic = jnp.sum(acc) * _SCALE
        obuf[...] = jnp.full((L,), ic, jnp.float32)
        pltpu.sync_copy(obuf, out_hbm.at[col])
        return 0

    lax.fori_loop(0, 2, lambda rep, c: do_column(wid + 32 * rep), 0)


@functools.partial(
    pl.kernel,
    out_type=jax.ShapeDtypeStruct((C, L), jnp.float32),
    mesh=plsc.VectorSubcoreMesh(core_axis_name="c", subcore_axis_name="s"),
    compiler_params=pltpu.CompilerParams(needs_layout_passes=False),
    scratch_types=[
        pltpu.VMEM((N,), jnp.int32),
        pltpu.VMEM((TWIN,), jnp.int32),
        pltpu.VMEM((MPAD,), jnp.int32),
        pltpu.VMEM((MPAD,), jnp.int32),
        pltpu.VMEM((MPAD,), jnp.int32),
        pltpu.VMEM((MPAD,), jnp.int32),
        pltpu.VMEM((L,), jnp.float32),
        pltpu.VMEM((2048,), jnp.int32),
        pltpu.VMEM((2048,), jnp.int32),
        pltpu.VMEM((2048,), jnp.int32),
        pltpu.VMEM((2048,), jnp.int32),
    ],
)
def _ic_sc(p_hbm, t_hbm, out_hbm, pbuf, twin, mk, mv, bk, bv, obuf,
           h0, h1, h2, h3):
    _ic_grid_kernel(p_hbm, t_hbm, out_hbm, pbuf, twin, mk, mv, bk, bv, obuf,
                    h0, h1, h2, h3)


def kernel(preds, targets):
    pT = lax.bitcast_convert_type(preds.T, jnp.int32)
    tT = lax.bitcast_convert_type(targets.T, jnp.int32)
    out = _ic_sc(pT, tT)
    return out[:, 0]


# speedup vs baseline: 20.5576x; 1.0819x over previous
import functools

import jax
import jax.numpy as jnp
from jax import lax
from jax.experimental import pallas as pl
from jax.experimental.pallas import tpu as pltpu
from jax.experimental.pallas import tpu_sc as plsc

N = 32768
C = 64
K = N // 2
L = 16
NV = N // L
KV = K // L
TWIN = 2048
NWIN = N // TWIN
MPAD = K + L

_VARK = K * (K + 1) / 12.0
_SCALE = float(1.0 / (K * (_VARK + 1e-8)))
_MHALF = (K - 1) / 2.0


def _isum(v):
    return jnp.sum(v.astype(jnp.int32))


def _f32(v):
    return v.astype(jnp.float32)


NQ = 8
KQ = KV // NQ


def _ic_grid_kernel(p_hbm, t_hbm, out_hbm, pbuf, twin, mk, mv, bk, bv, obuf,
                    *hists):
    cid = lax.axis_index("c")
    sid = lax.axis_index("s")
    wid = sid * 2 + cid
    ii = lax.iota(jnp.int32, L)
    hist = hists[0]

    def zero_hist(nbins):
        @plsc.parallel_loop(0, nbins // L, unroll=4)
        def _(ch):
            for q in range(NQ):
                hists[q][pl.ds(ch * L, L)] = jnp.zeros((L,), jnp.int32)

    def scan_hist(nbins, r):
        def sc(ch, carry):
            tot, bfound, cb = carry
            h = hist[pl.ds(ch * L, L)]
            cs = plsc.cumsum(h)
            cum = cs + tot
            found = cum >= r
            lane = _isum(jnp.where(found, jnp.int32(0), jnp.int32(1)))
            anyf = lane < L
            excl = cum - h
            cbc = _isum(jnp.where(ii == lane, excl, jnp.int32(0)))
            hit = (bfound < 0) & anyf
            bfound = jnp.where(hit, ch * L + lane, bfound)
            cb = jnp.where(hit, cbc, cb)
            return tot + _isum(h), bfound, cb
        _, b, cb = plsc.parallel_loop(
            0, nbins // L, unroll=2,
            carry=(jnp.int32(0), jnp.int32(-1), jnp.int32(0)))(sc)
        return b, cb

    def hist_pass(digit_fn, src):
        @plsc.parallel_loop(0, KQ, unroll=2)
        def _(v):
            for q in range(NQ):
                k = src[pl.ds((q * KQ + v) * L, L)]
                d = digit_fn(k)
                cnt, lastm = plsc.scan_count(d)
                plsc.addupdate_scatter(hists[q], [d], cnt, mask=lastm)

    def sel_pass(digit_fn, mask_fn):
        @plsc.parallel_loop(0, NV, unroll=4)
        def _(v):
            k = pbuf[pl.ds(v * L, L)]
            d = digit_fn(k)
            cnt, lastm = plsc.scan_count(d, mask=mask_fn(k))
            plsc.addupdate_scatter(hist, [d], cnt, mask=lastm)

    def exclusive_prefix(nbins):
        def pf(ch, tot):
            hv = [hists[q][pl.ds(ch * L, L)] for q in range(NQ)]
            h = hv[0]
            for q in range(1, NQ):
                h = h + hv[q]
            cs = plsc.cumsum(h)
            run = cs - h + (tot - 1)
            for q in range(NQ):
                hists[q][pl.ds(ch * L, L)] = run
                run = run + hv[q]
            return tot + _isum(h)
        plsc.parallel_loop(0, nbins // L, unroll=2, carry=jnp.int32(0))(pf)

    def radix_pass(nbins, digit_fn, src_k, dst_k, dst_v, src_v=None,
                   gen_val=None, fuse_cov=False):
        zero_hist(nbins)
        hist_pass(digit_fn, src_k)
        exclusive_prefix(nbins)

        def sp(v, acc):
            idx = [q * KQ + v for q in range(NQ)]
            k = [src_k[pl.ds(idx[q] * L, L)] for q in range(NQ)]
            if gen_val is not None:
                val = [gen_val(idx[q]) for q in range(NQ)]
            else:
                val = [src_v[pl.ds(idx[q] * L, L)] for q in range(NQ)]
            d = [digit_fn(k[q]) for q in range(NQ)]
            cl = [plsc.scan_count(d[q]) for q in range(NQ)]
            base = [plsc.load_gather(hists[q], [d[q]]) for q in range(NQ)]
            pos = [base[q] + cl[q][0] for q in range(NQ)]
            for q in range(NQ):
                plsc.addupdate_scatter(hists[q], [d[q]], cl[q][0],
                                       mask=cl[q][1])
            if fuse_cov:
                for q in range(NQ):
                    acc = acc + ((_MHALF - _f32(val[q]))
                                 * (_f32(pos[q]) - _MHALF))
            else:
                for q in range(NQ):
                    if dst_k is not None:
                        plsc.store_scatter(dst_k, [pos[q]], k[q])
                    plsc.store_scatter(dst_v, [pos[q]], val[q])
            return acc
        return lax.fori_loop(0, KQ, sp, jnp.zeros((L,), jnp.float32),
                             unroll=2)

    d_lo11 = lambda k: k & 0x7FF
    d_mid11 = lambda k: lax.shift_right_logical(k, 11) & 0x7FF
    d_top10 = lambda k: lax.shift_right_logical(k, 22) ^ 0x200

    def do_column(col):
        pltpu.sync_copy(p_hbm.at[col], pbuf)

        zero_hist(2048)

        @plsc.parallel_loop(0, NV, unroll=4)
        def _(v):
            x = pbuf[pl.ds(v * L, L)]
            sg = lax.shift_right_arithmetic(x, 31)
            key = ~(x ^ lax.shift_right_logical(sg, 1))
            pbuf[pl.ds(v * L, L)] = key
            d = lax.shift_right_logical(key, 21) ^ 0x400
            cnt, lastm = plsc.scan_count(d)
            plsc.addupdate_scatter(hist, [d], cnt, mask=lastm)

        b1, cb1 = scan_hist(2048, jnp.int32(K))
        r2 = K - cb1
        tb1 = (b1 ^ 0x400) << 21

        zero_hist(2048)
        sel_pass(
            lambda k: lax.shift_right_logical(k, 10) & 0x7FF,
            lambda k: lax.shift_right_logical(k, 21)
            == lax.shift_right_logical(tb1, 21))
        b2, cb2 = scan_hist(2048, r2)
        r3 = r2 - cb2
        tb2 = tb1 | (b2 << 10)

        zero_hist(1024)
        sel_pass(
            lambda k: k & 0x3FF,
            lambda k: lax.shift_right_logical(k, 10)
            == lax.shift_right_logical(tb2, 10))
        b3, cb3 = scan_hist(1024, r3)
        e_need = r3 - cb3
        tb = tb2 | b3

        def cw(w, carry):
            pltpu.sync_copy(t_hbm.at[col, pl.ds(w * TWIN, TWIN)], twin)

            def cv(v, carry2):
                wofs, eqs = carry2
                key = pbuf[pl.ds((w * (TWIN // L) + v) * L, L)]
                tx = twin[pl.ds(v * L, L)]
                ts = lax.shift_right_arithmetic(tx, 31)
                kt = tx ^ lax.shift_right_logical(ts, 1)
                lt = key < tb
                eq = key == tb
                ec = plsc.cumsum(jnp.where(eq, jnp.int32(1), jnp.int32(0)))
                take = eq & ((ec + eqs) <= e_need)
                member = lt | take
                plsc.store_compressed(mk.at[pl.ds(wofs, L)], key, mask=member)
                plsc.store_compressed(mv.at[pl.ds(wofs, L)], kt, mask=member)
                nlt = plsc.all_reduce_population_count(lt)[0]
                neq = plsc.all_reduce_population_count(eq)[0]
                ntake = jnp.clip(e_need - eqs, 0, neq)
                return wofs + nlt + ntake, eqs + neq
            return plsc.parallel_loop(0, TWIN // L, unroll=4, carry=carry)(cv)
        lax.fori_loop(0, NWIN, cw, (jnp.int32(0), jnp.int32(0)))

        radix_pass(2048, d_lo11, mk, bk, bv, src_v=mv)
        radix_pass(2048, d_mid11, bk, mk, mv, src_v=bv)
        radix_pass(1024, d_top10, mk, None, bv, src_v=mv)

        radix_pass(2048, d_lo11, bv, mk, mv, gen_val=lambda v: v * L + ii)
        radix_pass(2048, d_mid11, mk, bk, bv, src_v=mv)
        acc = radix_pass(1024, d_top10, bk, None, None, src_v=bv,
                         fuse_cov=True)

        ic = jnp.sum(acc) * _SCALE
        obuf[...] = jnp.full((L,), ic, jnp.float32)
        pltpu.sync_copy(obuf, out_hbm.at[col])
        return 0

    lax.fori_loop(0, 2, lambda rep, c: do_column(wid + 32 * rep), 0)


@functools.partial(
    pl.kernel,
    out_type=jax.ShapeDtypeStruct((C, L), jnp.float32),
    mesh=plsc.VectorSubcoreMesh(core_axis_name="c", subcore_axis_name="s"),
    compiler_params=pltpu.CompilerParams(needs_layout_passes=False),
    scratch_types=[
        pltpu.VMEM((N,), jnp.int32),
        pltpu.VMEM((TWIN,), jnp.int32),
        pltpu.VMEM((MPAD,), jnp.int32),
        pltpu.VMEM((MPAD,), jnp.int32),
        pltpu.VMEM((MPAD,), jnp.int32),
        pltpu.VMEM((MPAD,), jnp.int32),
        pltpu.VMEM((L,), jnp.float32),
    ] + [pltpu.VMEM((2048,), jnp.int32)] * NQ,
)
def _ic_sc(p_hbm, t_hbm, out_hbm, pbuf, twin, mk, mv, bk, bv, obuf, *hs):
    _ic_grid_kernel(p_hbm, t_hbm, out_hbm, pbuf, twin, mk, mv, bk, bv, obuf,
                    *hs)


def kernel(preds, targets):
    pT = lax.bitcast_convert_type(preds.T, jnp.int32)
    tT = lax.bitcast_convert_type(targets.T, jnp.int32)
    out = _ic_sc(pT, tT)
    return out[:, 0]


# speedup vs baseline: 20.5994x; 1.0020x over previous
import functools

import jax
import jax.numpy as jnp
from jax import lax
from jax.experimental import pallas as pl
from jax.experimental.pallas import tpu as pltpu
from jax.experimental.pallas import tpu_sc as plsc

N = 32768
C = 64
K = N // 2
L = 16
NV = N // L
KV = K // L
TWIN = 2048
NWIN = N // TWIN
MPAD = K + L

_VARK = K * (K + 1) / 12.0
_SCALE = float(1.0 / (K * (_VARK + 1e-8)))
_MHALF = (K - 1) / 2.0


def _isum(v):
    return jnp.sum(v.astype(jnp.int32))


def _f32(v):
    return v.astype(jnp.float32)


NQ = 4
KQ = KV // NQ


def _ic_grid_kernel(p_hbm, t_hbm, out_hbm, pbuf, twin, mk, mv, bk, bv, obuf,
                    cbuf, *hists):
    cid = lax.axis_index("c")
    sid = lax.axis_index("s")
    wid = sid * 2 + cid
    ii = lax.iota(jnp.int32, L)
    hist = hists[0]

    def zero_hist(nbins):
        @plsc.parallel_loop(0, nbins // L, unroll=4)
        def _(ch):
            for q in range(NQ):
                hists[q][pl.ds(ch * L, L)] = jnp.zeros((L,), jnp.int32)

    def scan_hist(nbins, r):
        def sc(ch, carry):
            tot, bfound, cb = carry
            h = hist[pl.ds(ch * L, L)]
            cs = plsc.cumsum(h)
            cum = cs + tot
            found = cum >= r
            lane = _isum(jnp.where(found, jnp.int32(0), jnp.int32(1)))
            anyf = lane < L
            excl = cum - h
            cbc = _isum(jnp.where(ii == lane, excl, jnp.int32(0)))
            hit = (bfound < 0) & anyf
            bfound = jnp.where(hit, ch * L + lane, bfound)
            cb = jnp.where(hit, cbc, cb)
            return tot + cs[15], bfound, cb
        _, b, cb = plsc.parallel_loop(
            0, nbins // L, unroll=2,
            carry=(jnp.int32(0), jnp.int32(-1), jnp.int32(0)))(sc)
        return b, cb

    def hist_pass(digit_fn, src):
        @plsc.parallel_loop(0, KQ, unroll=2)
        def _(v):
            for q in range(NQ):
                idx = q * KQ + v
                k = src[pl.ds(idx * L, L)]
                d = digit_fn(k)
                cnt, lastm = plsc.scan_count(d)
                cbuf[pl.ds(idx * L, L)] = cnt + jnp.where(
                    lastm, jnp.int32(32), jnp.int32(0))
                plsc.addupdate_scatter(hists[q], [d], cnt, mask=lastm)

    def sel_pass(digit_fn, mask_fn):
        @plsc.parallel_loop(0, NV, unroll=4)
        def _(v):
            k = pbuf[pl.ds(v * L, L)]
            d = digit_fn(k)
            cnt, lastm = plsc.scan_count(d, mask=mask_fn(k))
            plsc.addupdate_scatter(hist, [d], cnt, mask=lastm)

    def exclusive_prefix(nbins):
        def pf(ch, tot):
            hv = [hists[q][pl.ds(ch * L, L)] for q in range(NQ)]
            h = hv[0]
            for q in range(1, NQ):
                h = h + hv[q]
            cs = plsc.cumsum(h)
            run = cs - h + (tot - 1)
            for q in range(NQ):
                hists[q][pl.ds(ch * L, L)] = run
                run = run + hv[q]
            return tot + cs[15]
        plsc.parallel_loop(0, nbins // L, unroll=2, carry=jnp.int32(0))(pf)

    def radix_pass(nbins, digit_fn, src_k, dst_k, dst_v, src_v=None,
                   gen_val=None, fuse_cov=False):
        zero_hist(nbins)
        hist_pass(digit_fn, src_k)
        exclusive_prefix(nbins)

        def sp(v, acc):
            idx = [q * KQ + v for q in range(NQ)]
            k = [src_k[pl.ds(idx[q] * L, L)] for q in range(NQ)]
            if gen_val is not None:
                val = [gen_val(idx[q]) for q in range(NQ)]
            else:
                val = [src_v[pl.ds(idx[q] * L, L)] for q in range(NQ)]
            d = [digit_fn(k[q]) for q in range(NQ)]
            cp = [cbuf[pl.ds(idx[q] * L, L)] for q in range(NQ)]
            cl = [(cp[q] & 31, cp[q] > 31) for q in range(NQ)]
            base = [plsc.load_gather(hists[q], [d[q]]) for q in range(NQ)]
            pos = [base[q] + cl[q][0] for q in range(NQ)]
            for q in range(NQ):
                plsc.addupdate_scatter(hists[q], [d[q]], cl[q][0],
                                       mask=cl[q][1])
            if fuse_cov:
                for q in range(NQ):
                    acc = acc + ((_MHALF - _f32(val[q]))
                                 * (_f32(pos[q]) - _MHALF))
            else:
                for q in range(NQ):
                    if dst_k is not None:
                        plsc.store_scatter(dst_k, [pos[q]], k[q])
                    plsc.store_scatter(dst_v, [pos[q]], val[q])
            return acc
        return lax.fori_loop(0, KQ, sp, jnp.zeros((L,), jnp.float32),
                             unroll=2)

    d_lo11 = lambda k: k & 0x7FF
    d_mid11 = lambda k: lax.shift_right_logical(k, 11) & 0x7FF
    d_top10 = lambda k: lax.shift_right_logical(k, 22) ^ 0x200

    def do_column(col):
        pltpu.sync_copy(p_hbm.at[col], pbuf)

        zero_hist(2048)

        @plsc.parallel_loop(0, NV, unroll=4)
        def _(v):
            x = pbuf[pl.ds(v * L, L)]
            sg = lax.shift_right_arithmetic(x, 31)
            key = ~(x ^ lax.shift_right_logical(sg, 1))
            pbuf[pl.ds(v * L, L)] = key
            d = lax.shift_right_logical(key, 21) ^ 0x400
            cnt, lastm = plsc.scan_count(d)
            plsc.addupdate_scatter(hist, [d], cnt, mask=lastm)

        b1, cb1 = scan_hist(2048, jnp.int32(K))
        r2 = K - cb1
        tb1 = (b1 ^ 0x400) << 21

        zero_hist(2048)
        sel_pass(
            lambda k: lax.shift_right_logical(k, 10) & 0x7FF,
            lambda k: lax.shift_right_logical(k, 21)
            == lax.shift_right_logical(tb1, 21))
        b2, cb2 = scan_hist(2048, r2)
        r3 = r2 - cb2
        tb2 = tb1 | (b2 << 10)

        zero_hist(1024)
        sel_pass(
            lambda k: k & 0x3FF,
            lambda k: lax.shift_right_logical(k, 10)
            == lax.shift_right_logical(tb2, 10))
        b3, cb3 = scan_hist(1024, r3)
        e_need = r3 - cb3
        tb = tb2 | b3

        def cw(w, carry):
            pltpu.sync_copy(t_hbm.at[col, pl.ds(w * TWIN, TWIN)], twin)

            def cv(v, carry2):
                wofs, eqs = carry2
                key = pbuf[pl.ds((w * (TWIN // L) + v) * L, L)]
                tx = twin[pl.ds(v * L, L)]
                ts = lax.shift_right_arithmetic(tx, 31)
                kt = tx ^ lax.shift_right_logical(ts, 1)
                lt = key < tb
                eq = key == tb
                ec = plsc.cumsum(jnp.where(eq, jnp.int32(1), jnp.int32(0)))
                take = eq & ((ec + eqs) <= e_need)
                member = lt | take
                plsc.store_compressed(mk.at[pl.ds(wofs, L)], key, mask=member)
                plsc.store_compressed(mv.at[pl.ds(wofs, L)], kt, mask=member)
                nlt = plsc.all_reduce_population_count(lt)[0]
                neq = plsc.all_reduce_population_count(eq)[0]
                ntake = jnp.clip(e_need - eqs, 0, neq)
                return wofs + nlt + ntake, eqs + neq
            return plsc.parallel_loop(0, TWIN // L, unroll=4, carry=carry)(cv)
        lax.fori_loop(0, NWIN, cw, (jnp.int32(0), jnp.int32(0)))

        radix_pass(2048, d_lo11, mk, bk, bv, src_v=mv)
        radix_pass(2048, d_mid11, bk, mk, mv, src_v=bv)
        radix_pass(1024, d_top10, mk, None, bv, src_v=mv)

        radix_pass(2048, d_lo11, bv, mk, mv, gen_val=lambda v: v * L + ii)
        radix_pass(2048, d_mid11, mk, bk, bv, src_v=mv)
        acc = radix_pass(1024, d_top10, bk, None, None, src_v=bv,
                         fuse_cov=True)

        ic = jnp.sum(acc) * _SCALE
        obuf[...] = jnp.full((L,), ic, jnp.float32)
        pltpu.sync_copy(obuf, out_hbm.at[col])
        return 0

    lax.fori_loop(0, 2, lambda rep, c: do_column(wid + 32 * rep), 0)


@functools.partial(
    pl.kernel,
    out_type=jax.ShapeDtypeStruct((C, L), jnp.float32),
    mesh=plsc.VectorSubcoreMesh(core_axis_name="c", subcore_axis_name="s"),
    compiler_params=pltpu.CompilerParams(needs_layout_passes=False),
    scratch_types=[
        pltpu.VMEM((N,), jnp.int32),
        pltpu.VMEM((TWIN,), jnp.int32),
        pltpu.VMEM((MPAD,), jnp.int32),
        pltpu.VMEM((MPAD,), jnp.int32),
        pltpu.VMEM((MPAD,), jnp.int32),
        pltpu.VMEM((MPAD,), jnp.int32),
        pltpu.VMEM((L,), jnp.float32),
        pltpu.VMEM((K,), jnp.int32),
    ] + [pltpu.VMEM((2048,), jnp.int32)] * NQ,
)
def _ic_sc(p_hbm, t_hbm, out_hbm, pbuf, twin, mk, mv, bk, bv, obuf, cbuf,
           *hs):
    _ic_grid_kernel(p_hbm, t_hbm, out_hbm, pbuf, twin, mk, mv, bk, bv, obuf,
                    cbuf, *hs)


def kernel(preds, targets):
    pT = lax.bitcast_convert_type(preds.T, jnp.int32)
    tT = lax.bitcast_convert_type(targets.T, jnp.int32)
    out = _ic_sc(pT, tT)
    return out[:, 0]
